# baseline (speedup 1.0000x reference)
import functools

import jax
import jax.numpy as jnp
from jax import lax
from jax.experimental import pallas as pl
from jax.experimental.pallas import tpu as pltpu
from jax.experimental.pallas import tpu_sc as plsc

N_KEYS = 100000
BITS = 512
SPLIT = 96000

ROWS = 4000
NB = SPLIT // ROWS
CH = ROWS // 128
TAIL = ROWS - CH * 128

SC_ROWS = N_KEYS - SPLIT
CHUNK = 80
CID0 = SPLIT // CHUNK
NCH = N_KEYS // CHUNK
NW = 32
TT = -(-(NCH - CID0) // NW)
TT += TT % 2
NJ = BITS // 16


def _tc_body(t_ref, w_ref, k_ref, o_ref):
    kb = k_ref[...].astype(jnp.bfloat16)
    d = jax.lax.dot_general(
        kb, w_ref[...], (((1,), (0,)), ((), ())),
        preferred_element_type=jnp.float32)
    d3 = d[:CH * 128].reshape(CH, 128, 128)
    row_i = jax.lax.broadcasted_iota(jnp.int32, (128, 128), 0)
    col_i = jax.lax.broadcasted_iota(jnp.int32, (128, 128), 1)
    eye = jnp.where(row_i == col_i, 1.0, 0.0)
    diag = jnp.sum(d3 * eye[None], axis=1)
    diag_tail = jnp.sum(d[CH * 128:] * eye[:TAIL], axis=0)
    t = t_ref[0]
    w_main = jnp.where(diag <= t, 1.0, 0.0).reshape(CH * 128)
    w_tail = jnp.where(diag_tail <= t, 1.0, 0.0)[:TAIL]
    o_ref[...] = jnp.concatenate([w_main, w_tail]).reshape(1, 1, ROWS)


def _tc_call(t, wmat, keys):
    return pl.pallas_call(
        _tc_body,
        grid=(NB,),
        in_specs=[
            pl.BlockSpec(memory_space=pltpu.SMEM),
            pl.BlockSpec((BITS, 128), lambda i: (0, 0)),
            pl.BlockSpec((ROWS, BITS), lambda i: (i, 0)),
        ],
        out_specs=pl.BlockSpec((1, 1, ROWS), lambda i: (i, 0, 0)),
        out_shape=jax.ShapeDtypeStruct((NB, 1, ROWS), jnp.float32),
    )(t, wmat, keys).reshape(SPLIT)


def _sc_body(thr_hbm, w_hbm, keys_hbm, out_hbm, buf, wv, tv, outv,
             sem0, sem1):
    c = lax.axis_index("c")
    s = lax.axis_index("s")
    wid = s * 2 + c
    pltpu.sync_copy(w_hbm, wv)
    pltpu.sync_copy(thr_hbm, tv)
    tvec = tv[pl.ds(0, 16)]
    wregs = [wv[pl.ds(16 * j, 16)] for j in range(NJ)]
    lane = lax.broadcasted_iota(jnp.int32, (16,), 0)
    sems = (sem0, sem1)

    def start(ti, b):
        cid = CID0 + wid + NW * ti

        @pl.when(cid < NCH)
        def _():
            pltpu.async_copy(
                keys_hbm.at[pl.ds(cid * CHUNK, CHUNK)], buf.at[b], sems[b])

    def row_acc(b, r):
        acc = wregs[0] * buf[b, r, pl.ds(0, 16)]
        for j in range(1, NJ):
            acc = acc + wregs[j] * buf[b, r, pl.ds(16 * j, 16)]
        return acc

    def tree_step(x, y, st):
        m = (lane & st) != 0
        u = jnp.where(m, y, x)
        v = jnp.where(m, x, y)
        return u + v[lane ^ st]

    BITREV = (0, 8, 4, 12, 2, 10, 6, 14, 1, 9, 5, 13, 3, 11, 7, 15)

    def process(ti, b):
        cid = CID0 + wid + NW * ti

        @pl.when(cid < NCH)
        def _():
            pltpu.make_async_copy(
                keys_hbm.at[pl.ds(cid * CHUNK, CHUNK)], buf.at[b],
                sems[b]).wait()

            def groupfn(g, carry):
                r0 = 16 * g
                vecs = [row_acc(b, r0 + BITREV[i]) for i in range(16)]
                for st in (8, 4, 2, 1):
                    vecs = [tree_step(vecs[2 * i], vecs[2 * i + 1], st)
                            for i in range(len(vecs) // 2)]
                dvec = vecs[0]
                outv[pl.ds(r0, 16)] = jnp.where(dvec <= tvec, 1.0, 0.0)
                return carry

            lax.fori_loop(0, CHUNK // 16, groupfn, 0)
            pltpu.sync_copy(
                outv, out_hbm.at[pl.ds(cid * CHUNK - SPLIT, CHUNK)])

    start(0, 0)

    def pair(p, carry):
        for bb in (0, 1):
            ti = 2 * p + bb
            start(ti + 1, 1 - bb)
            process(ti, bb)
        return carry

    lax.fori_loop(0, TT // 2, pair, 0)


def _sc_call(thr, w, keys):
    mesh = plsc.VectorSubcoreMesh(core_axis_name="c", subcore_axis_name="s")
    kfn = functools.partial(
        pl.kernel, mesh=mesh,
        out_type=jax.ShapeDtypeStruct((SC_ROWS,), jnp.float32),
        scratch_types=[
            pltpu.VMEM((2, CHUNK, BITS), jnp.float32),
            pltpu.VMEM((BITS,), jnp.float32),
            pltpu.VMEM((16,), jnp.float32),
            pltpu.VMEM((CHUNK,), jnp.float32),
            pltpu.SemaphoreType.DMA,
            pltpu.SemaphoreType.DMA,
        ])(_sc_body)
    return kfn(thr, w, keys)


def kernel(query, keys):
    q = jnp.reshape(query, (BITS,))
    wf = 1.0 - 2.0 * q
    w = wf.astype(jnp.bfloat16)
    wmat = jnp.tile(w[:, None], (1, 128))
    t = (1.0 - jnp.sum(q)).reshape(1)
    thr = jnp.full((16,), 1.0 - jnp.sum(q), jnp.float32)
    sc_out = _sc_call(thr, wf, keys)
    tc_out = _tc_call(t, wmat, keys)
    return jnp.concatenate([tc_out, sc_out])


# speedup vs baseline: 1.3074x; 1.3074x over previous
import jax
import jax.numpy as jnp
from jax.experimental import pallas as pl
from jax.experimental.pallas import tpu as pltpu

N_KEYS = 100000
BITS = 512
ROWS = 10000
NB = N_KEYS // ROWS
CH = ROWS // 128
TAIL = ROWS - CH * 128


def _body(t_ref, w_ref, k_ref, o_ref):
    kb = k_ref[...].astype(jnp.bfloat16)
    d = jax.lax.dot_general(
        kb, w_ref[...], (((1,), (0,)), ((), ())),
        preferred_element_type=jnp.float32)
    d3 = d[:CH * 128].reshape(CH, 128, 128)
    row_i = jax.lax.broadcasted_iota(jnp.int32, (128, 128), 0)
    col_i = jax.lax.broadcasted_iota(jnp.int32, (128, 128), 1)
    eye = jnp.where(row_i == col_i, 1.0, 0.0)
    diag = jnp.sum(d3 * eye[None], axis=1)
    diag_tail = jnp.sum(d[CH * 128:] * eye[:TAIL], axis=0)
    t = t_ref[0]
    w_main = jnp.where(diag <= t, 1.0, 0.0).reshape(CH * 128)
    w_tail = jnp.where(diag_tail <= t, 1.0, 0.0)[:TAIL]
    o_ref[...] = jnp.concatenate([w_main, w_tail]).reshape(1, 1, ROWS)


def kernel(query, keys):
    q = jnp.reshape(query, (BITS,))
    w = (1.0 - 2.0 * q).astype(jnp.bfloat16)
    wmat = jnp.tile(w[:, None], (1, 128))
    t = (1.0 - jnp.sum(q)).reshape(1)
    return pl.pallas_call(
        _body,
        grid=(NB,),
        in_specs=[
            pl.BlockSpec(memory_space=pltpu.SMEM),
            pl.BlockSpec((BITS, 128), lambda i: (0, 0)),
            pl.BlockSpec((ROWS, BITS), lambda i: (i, 0)),
        ],
        out_specs=pl.BlockSpec((1, 1, ROWS), lambda i: (i, 0, 0)),
        out_shape=jax.ShapeDtypeStruct((NB, 1, ROWS), jnp.float32),
    )(t, wmat, keys).reshape(N_KEYS)
